# Initial kernel scaffold; baseline (speedup 1.0000x reference)
import jax
import jax.numpy as jnp
from jax.experimental import pallas as pl


def kernel(temporal_data, revert_idx, temporal_pos_enc, remain_padding_mask, mask_token):
    raise NotImplementedError("write your pallas kernel here")


# speedup vs baseline: 2.7898x; 2.7898x over previous
import functools

import jax
import jax.numpy as jnp
from jax import lax
from jax.experimental import pallas as pl
from jax.experimental.pallas import tpu as pltpu
from jax.experimental.pallas import tpu_sc as plsc

B = 4
L_REMAIN = 2048
D = 1024
N = 8192
LFULL = N + 1
MASK_ROW = B * L_REMAIN
CR = 16
QF = N // CR
Q = QF + 1
T = B * Q
NC, NS = 2, 16
NW = NC * NS
OUTER = (T + NW - 1) // NW


def _sc_revert(table, ridx_flat, pos_enc, pm_flat):
    mesh = plsc.VectorSubcoreMesh(core_axis_name="c", subcore_axis_name="s")

    @functools.partial(
        pl.kernel,
        out_type=jax.ShapeDtypeStruct((B, LFULL, D), jnp.float32),
        mesh=mesh,
        compiler_params=pltpu.CompilerParams(needs_layout_passes=False),
        scratch_types=[
            pltpu.VMEM((B * N,), jnp.int32),
            pltpu.VMEM((B * L_REMAIN,), jnp.int32),
            pltpu.VMEM((CR,), jnp.int32),
            pltpu.VMEM((CR, D), jnp.float32),
            pltpu.VMEM((CR, D), jnp.float32),
            pltpu.SemaphoreType.DMA,
            pltpu.SemaphoreType.DMA,
        ],
    )
    def k(table_hbm, ridx_hbm, pos_hbm, pm_hbm, out_hbm,
          ridx_v, pm_v, idx_v, rows_v, pos_v, sem1, sem2):
        wid = lax.axis_index("s") * NC + lax.axis_index("c")
        pltpu.sync_copy(ridx_hbm, ridx_v)
        pltpu.sync_copy(pm_hbm, pm_v)
        lanes = lax.iota(jnp.int32, CR)

        def compute_src(b, ivec):
            fr = jnp.clip(b * N + ivec - 1, 0, B * N - 1)
            r = plsc.load_gather(ridx_v, [fr])
            j = r + 1
            in_rng = (ivec > 0) & (ivec <= N) & (j <= L_REMAIN - 1)
            fp = jnp.clip(b * L_REMAIN + j - 1, 0, B * L_REMAIN - 1)
            pmv = plsc.load_gather(pm_v, [fp])
            valid = in_rng & (pmv == 1)
            return jnp.where(valid, b * L_REMAIN + j, MASK_ROW)

        def add_rows(nrows):
            def add_body(rr, carry2):
                for kk in range(D // 16):
                    sl = pl.ds(kk * 16, 16)
                    plsc.addupdate(rows_v.at[rr, sl], pos_v[rr, sl])
                return carry2
            lax.fori_loop(0, nrows, add_body, 0)

        def chunk_body(c, carry):
            t = c * NW + wid
            b = t // Q
            q = t - b * Q

            @pl.when((t < T) & (q < QF))
            def _():
                i0 = pl.multiple_of(q * CR, CR)
                idx_v[...] = compute_src(b, i0 + lanes)
                cp1 = pltpu.async_copy(table_hbm.at[idx_v], rows_v, sem1)
                cp2 = pltpu.async_copy(pos_hbm.at[pl.ds(i0, CR), :], pos_v, sem2)
                cp1.wait()
                cp2.wait()
                add_rows(CR)
                pltpu.sync_copy(rows_v, out_hbm.at[b, pl.ds(i0, CR), :])

            @pl.when((t < T) & (q == QF))
            def _():
                idx_v[...] = compute_src(b, jnp.full((CR,), N, jnp.int32))
                cp1 = pltpu.async_copy(table_hbm.at[idx_v], rows_v, sem1)
                cp2 = pltpu.async_copy(pos_hbm.at[pl.ds(N, CR), :], pos_v, sem2)
                cp1.wait()
                cp2.wait()
                add_rows(1)
                pltpu.sync_copy(rows_v.at[pl.ds(0, 1), :],
                                out_hbm.at[b, pl.ds(N, 1), :])

            return carry

        lax.fori_loop(0, OUTER, chunk_body, 0)

    return k(table, ridx_flat, pos_enc, pm_flat)


def kernel(temporal_data, revert_idx, temporal_pos_enc, remain_padding_mask, mask_token):
    table = jnp.concatenate(
        [temporal_data.reshape(B * L_REMAIN, D), mask_token], axis=0)
    ridx_flat = revert_idx.reshape(B * N)
    pm_flat = jnp.pad(remain_padding_mask, ((0, 0), (0, 1))).reshape(B * L_REMAIN)
    return _sc_revert(table, ridx_flat, temporal_pos_enc, pm_flat)


# speedup vs baseline: 2.8567x; 1.0240x over previous
import functools

import jax
import jax.numpy as jnp
from jax import lax
from jax.experimental import pallas as pl
from jax.experimental.pallas import tpu as pltpu
from jax.experimental.pallas import tpu_sc as plsc

B = 4
L_REMAIN = 2048
D = 1024
N = 8192
LFULL = N + 1
MASK_ROW = B * L_REMAIN
IR = 8
CRW = B * IR
QN = N // IR
NC, NS = 2, 16
NW = NC * NS
CPW = QN // NW
HALF = (CPW + 1) // 2


def _sc_revert(table, ridx_flat, pos_enc, pm_flat):
    mesh = plsc.VectorSubcoreMesh(core_axis_name="c", subcore_axis_name="s")

    @functools.partial(
        pl.kernel,
        out_type=jax.ShapeDtypeStruct((B, LFULL, D), jnp.float32),
        mesh=mesh,
        compiler_params=pltpu.CompilerParams(needs_layout_passes=False),
        scratch_types=[
            pltpu.VMEM((B * N,), jnp.int32),
            pltpu.VMEM((B * L_REMAIN,), jnp.int32),
            pltpu.VMEM((CRW,), jnp.int32),
            pltpu.VMEM((CRW,), jnp.int32),
            pltpu.VMEM((CRW, D), jnp.float32),
            pltpu.VMEM((CRW, D), jnp.float32),
            pltpu.VMEM((IR, D), jnp.float32),
            pltpu.VMEM((IR, D), jnp.float32),
            pltpu.SemaphoreType.DMA,
            pltpu.SemaphoreType.DMA,
            pltpu.SemaphoreType.DMA,
            pltpu.SemaphoreType.DMA,
            pltpu.SemaphoreType.DMA,
            pltpu.SemaphoreType.DMA,
        ],
    )
    def k(table_hbm, ridx_hbm, pos_hbm, pm_hbm, out_hbm,
          ridx_v, pm_v, idx0, idx1, rows0, rows1, pos0, pos1,
          gsem0, gsem1, psem0, psem1, wsem0, wsem1):
        idx_s = (idx0, idx1)
        rows_s = (rows0, rows1)
        pos_s = (pos0, pos1)
        gsem_s = (gsem0, gsem1)
        psem_s = (psem0, psem1)
        wsem_s = (wsem0, wsem1)

        wid = lax.axis_index("s") * NC + lax.axis_index("c")
        pltpu.sync_copy(ridx_hbm, ridx_v)
        pltpu.sync_copy(pm_hbm, pm_v)
        lanes = lax.iota(jnp.int32, 16)

        def compute_src(bv, ivec):
            fr = jnp.clip(bv * N + ivec - 1, 0, B * N - 1)
            r = plsc.load_gather(ridx_v, [fr])
            j = r + 1
            in_rng = (ivec > 0) & (ivec <= N) & (j <= L_REMAIN - 1)
            fp = jnp.clip(bv * L_REMAIN + j - 1, 0, B * L_REMAIN - 1)
            pmv = plsc.load_gather(pm_v, [fp])
            valid = in_rng & (pmv == 1)
            return jnp.where(valid, bv * L_REMAIN + j, MASK_ROW)

        def chunk_i0(n):
            return pl.multiple_of((wid * CPW + n) * IR, IR)

        def start(n, s):
            i0 = chunk_i0(n)
            for h in range(2):
                flat = lanes + 16 * h
                bv = flat // IR
                ivec = i0 + (flat - bv * IR)
                idx_s[s][pl.ds(16 * h, 16)] = compute_src(bv, ivec)
            pltpu.async_copy(table_hbm.at[idx_s[s]], rows_s[s], gsem_s[s])
            pltpu.async_copy(pos_hbm.at[pl.ds(i0, IR), :], pos_s[s], psem_s[s])

        def finish(n, s):
            i0 = chunk_i0(n)
            pltpu.make_async_copy(table_hbm.at[idx_s[s]], rows_s[s],
                                  gsem_s[s]).wait()
            pltpu.make_async_copy(pos_hbm.at[pl.ds(i0, IR), :], pos_s[s],
                                  psem_s[s]).wait()

            def add_body(rr, carry2):
                il = rr - (rr // IR) * IR
                for kk in range(D // 16):
                    sl = pl.ds(kk * 16, 16)
                    plsc.addupdate(rows_s[s].at[rr, sl], pos_s[s][il, sl])
                return carry2

            lax.fori_loop(0, CRW, add_body, 0)
            for b in range(B):
                pltpu.async_copy(rows_s[s].at[pl.ds(IR * b, IR), :],
                                 out_hbm.at[b, pl.ds(i0, IR), :], wsem_s[s])

        def wait_writes(n, s):
            i0 = chunk_i0(n)
            for b in range(B):
                pltpu.make_async_copy(rows_s[s].at[pl.ds(IR * b, IR), :],
                                      out_hbm.at[b, pl.ds(i0, IR), :],
                                      wsem_s[s]).wait()

        def body(n, carry):
            start(n, 0)
            finish(n, 0)
            wait_writes(n, 0)
            return carry

        lax.fori_loop(0, CPW, body, 0)

        @pl.when(wid < B)
        def _():
            bt = wid
            src = compute_src(jnp.full((16,), bt, jnp.int32),
                              jnp.full((16,), N, jnp.int32))
            idx0[pl.ds(0, 16)] = src
            idx0[pl.ds(16, 16)] = src
            pltpu.async_copy(table_hbm.at[idx0], rows0, gsem0).wait()
            pltpu.async_copy(pos_hbm.at[pl.ds(N, IR), :], pos0, psem0).wait()
            for kk in range(D // 16):
                sl = pl.ds(kk * 16, 16)
                plsc.addupdate(rows0.at[0, sl], pos0[0, sl])
            pltpu.sync_copy(rows0.at[pl.ds(0, 1), :],
                            out_hbm.at[bt, pl.ds(N, 1), :])

    return k(table, ridx_flat, pos_enc, pm_flat)


def kernel(temporal_data, revert_idx, temporal_pos_enc, remain_padding_mask, mask_token):
    table = jnp.concatenate(
        [temporal_data.reshape(B * L_REMAIN, D), mask_token], axis=0)
    ridx_flat = revert_idx.reshape(B * N)
    pm_flat = jnp.pad(remain_padding_mask, ((0, 0), (0, 1))).reshape(B * L_REMAIN)
    return _sc_revert(table, ridx_flat, temporal_pos_enc, pm_flat)
